# Initial kernel scaffold; baseline (speedup 1.0000x reference)
import jax
import jax.numpy as jnp
from jax.experimental import pallas as pl


def kernel(x, adj, Wl, Wr, att, bias):
    raise NotImplementedError("write your pallas kernel here")


# speedup vs baseline: 3.1685x; 3.1685x over previous
import jax
import jax.numpy as jnp
from jax import lax
from jax.experimental import pallas as pl

_NEG = -1e30


def _gat_body(x_ref, xt_ref, adj_ref, wl_ref, wrt_ref, att_ref, attc_ref,
              bias_ref, out_ref):
    n = x_ref.shape[1]
    dh = wl_ref.shape[2]
    x = x_ref[0]
    xt = xt_ref[0]
    wl = wl_ref[0]
    wrt = wrt_ref[0]
    att = att_ref[0]
    attc = attc_ref[0]

    xl = jnp.dot(x, wl, preferred_element_type=jnp.float32)
    xrat = jnp.dot(wrt, xt, preferred_element_type=jnp.float32)

    xla = xl * att
    al = jnp.sum(xla, axis=1, keepdims=True)
    ar = jnp.sum(xrat * attc, axis=0, keepdims=True)

    q = jnp.zeros((n, n), jnp.float32)
    for d in range(dh):
        a_d = att[0, d]
        t = xla[:, d:d + 1] + xrat[d:d + 1, :] * a_d
        q = q + jnp.abs(t) * jnp.sign(a_d)
    s = 0.6 * (al + ar) + 0.4 * q

    m = adj_ref[0] != 0
    s = jnp.where(m, s, _NEG)
    amax = jnp.max(s, axis=0, keepdims=True)
    amax = jnp.where(amax > 0.5 * _NEG, amax, 0.0)
    ex = jnp.exp(s - amax)
    denom = jnp.sum(ex, axis=0, keepdims=True) + 1e-16
    alpha = ex * (1.0 / denom)

    out = lax.dot_general(alpha, xl, (((0,), (0,)), ((), ())),
                          preferred_element_type=jnp.float32)
    out_ref[0, 0] = out + bias_ref[0]


def kernel(x, adj, Wl, Wr, att, bias):
    b, n, in_dim = x.shape
    heads, dh = att.shape

    xt = x.transpose(0, 2, 1)
    adj8 = (adj != 0).astype(jnp.int8)
    wl = Wl.reshape(in_dim, heads, dh).transpose(1, 0, 2)
    wrt = Wr.reshape(in_dim, heads, dh).transpose(1, 2, 0)
    attr = att.reshape(heads, 1, dh)
    attc = att.reshape(heads, dh, 1)
    biasr = bias.reshape(heads, 1, dh)

    out = pl.pallas_call(
        _gat_body,
        grid=(b, heads),
        in_specs=[
            pl.BlockSpec((1, n, in_dim), lambda bb, h: (bb, 0, 0)),
            pl.BlockSpec((1, in_dim, n), lambda bb, h: (bb, 0, 0)),
            pl.BlockSpec((1, n, n), lambda bb, h: (bb, 0, 0)),
            pl.BlockSpec((1, in_dim, dh), lambda bb, h: (h, 0, 0)),
            pl.BlockSpec((1, dh, in_dim), lambda bb, h: (h, 0, 0)),
            pl.BlockSpec((1, 1, dh), lambda bb, h: (h, 0, 0)),
            pl.BlockSpec((1, dh, 1), lambda bb, h: (h, 0, 0)),
            pl.BlockSpec((1, 1, dh), lambda bb, h: (h, 0, 0)),
        ],
        out_specs=pl.BlockSpec((1, 1, n, dh), lambda bb, h: (bb, h, 0, 0)),
        out_shape=jax.ShapeDtypeStruct((b, heads, n, dh), jnp.float32),
    )(x, xt, adj8, wl, wrt, attr, attc, biasr)

    return out.transpose(0, 2, 1, 3).reshape(b, n, heads * dh)


# speedup vs baseline: 3.1702x; 1.0005x over previous
import jax
import jax.numpy as jnp
from jax import lax
from jax.experimental import pallas as pl
from jax.experimental.pallas import tpu as pltpu

_NEG = -1e30


def _gat_body(x_ref, xt_ref, adj_ref, wl_ref, wrt_ref, att_ref, attc_ref,
              bias_ref, out_ref):
    n = x_ref.shape[1]
    dh = wl_ref.shape[2]
    x = x_ref[0]
    xt = xt_ref[0]
    wl = wl_ref[0]
    wrt = wrt_ref[0]
    att = att_ref[0]
    attc = attc_ref[0]

    xl = jnp.dot(x, wl, preferred_element_type=jnp.float32)
    xrat = jnp.dot(wrt, xt, preferred_element_type=jnp.float32)

    xla = xl * att
    al = jnp.sum(xla, axis=1, keepdims=True)
    ar = jnp.sum(xrat * attc, axis=0, keepdims=True)

    q = jnp.zeros((n, n), jnp.float32)
    for d in range(dh):
        a_d = att[0, d]
        t = xla[:, d:d + 1] + xrat[d:d + 1, :] * a_d
        q = q + jnp.abs(t) * jnp.sign(a_d)
    s = 0.6 * (al + ar) + 0.4 * q

    m = adj_ref[0] != 0
    s = jnp.where(m, s, _NEG)
    amax = jnp.max(s, axis=0, keepdims=True)
    amax = jnp.where(amax > 0.5 * _NEG, amax, 0.0)
    ex = jnp.exp(s - amax)
    denom = jnp.sum(ex, axis=0, keepdims=True) + 1e-16
    alpha = ex * (1.0 / denom)

    out = lax.dot_general(alpha, xl, (((0,), (0,)), ((), ())),
                          preferred_element_type=jnp.float32)
    out_ref[0, 0] = out + bias_ref[0]


def kernel(x, adj, Wl, Wr, att, bias):
    b, n, in_dim = x.shape
    heads, dh = att.shape

    xt = x.transpose(0, 2, 1)
    adj8 = (adj != 0).astype(jnp.int8)
    wl = Wl.reshape(in_dim, heads, dh).transpose(1, 0, 2)
    wrt = Wr.reshape(in_dim, heads, dh).transpose(1, 2, 0)
    attr = att.reshape(heads, 1, dh)
    attc = att.reshape(heads, dh, 1)
    biasr = bias.reshape(heads, 1, dh)

    out = pl.pallas_call(
        _gat_body,
        grid=(b, heads),
        in_specs=[
            pl.BlockSpec((1, n, in_dim), lambda bb, h: (bb, 0, 0)),
            pl.BlockSpec((1, in_dim, n), lambda bb, h: (bb, 0, 0)),
            pl.BlockSpec((1, n, n), lambda bb, h: (bb, 0, 0)),
            pl.BlockSpec((1, in_dim, dh), lambda bb, h: (h, 0, 0)),
            pl.BlockSpec((1, dh, in_dim), lambda bb, h: (h, 0, 0)),
            pl.BlockSpec((1, 1, dh), lambda bb, h: (h, 0, 0)),
            pl.BlockSpec((1, dh, 1), lambda bb, h: (h, 0, 0)),
            pl.BlockSpec((1, 1, dh), lambda bb, h: (h, 0, 0)),
        ],
        out_specs=pl.BlockSpec((1, 1, n, dh), lambda bb, h: (bb, h, 0, 0)),
        out_shape=jax.ShapeDtypeStruct((b, heads, n, dh), jnp.float32),
        compiler_params=pltpu.CompilerParams(
            dimension_semantics=("parallel", "parallel")),
    )(x, xt, adj8, wl, wrt, attr, attc, biasr)

    return out.transpose(0, 2, 1, 3).reshape(b, n, heads * dh)


# speedup vs baseline: 4.2915x; 1.3537x over previous
import jax
import jax.numpy as jnp
from jax import lax
from jax.experimental import pallas as pl
from jax.experimental.pallas import tpu as pltpu

_NEG = -1e30
_TJ = 128
_TI = 128


def _gat_body(x_ref, xt_ref, adj_ref, wl_ref, wlt_ref, wrt_ref, att_ref,
              attc_ref, bias_ref, out_ref, s_scr):
    n = x_ref.shape[1]
    dh = wl_ref.shape[2]
    x = x_ref[0]
    xt = xt_ref[0]
    wl = wl_ref[0]
    wlt = wlt_ref[0]
    wrt = wrt_ref[0]
    att = att_ref[0]
    attc = attc_ref[0]

    xl = jnp.dot(x, wl, preferred_element_type=jnp.float32)
    xlt = jnp.dot(wlt, xt, preferred_element_type=jnp.float32)
    xrat = jnp.dot(wrt, xt, preferred_element_type=jnp.float32)

    xla = xl * (0.4 * att)
    xrab = xrat * (0.4 * attc)
    al2 = 1.5 * jnp.sum(xla, axis=1, keepdims=True)
    ar2 = 1.5 * jnp.sum(xrab, axis=0, keepdims=True)

    nj = n // _TJ
    ni = n // _TI

    pmax = []
    for it in range(ni):
        ii = it * _TI
        pm = None
        for jt in range(nj):
            jj = jt * _TJ
            acc = al2[jj:jj + _TJ] + ar2[:, ii:ii + _TI]
            for d in range(dh):
                t = xla[jj:jj + _TJ, d:d + 1] + xrab[d:d + 1, ii:ii + _TI]
                acc = acc + jnp.abs(t) * jnp.sign(att[0, d])
            m = adj_ref[0, jj:jj + _TJ, ii:ii + _TI] != 0
            acc = jnp.where(m, acc, _NEG)
            s_scr[jj:jj + _TJ, ii:ii + _TI] = acc
            t_pm = jnp.max(acc, axis=0, keepdims=True)
            pm = t_pm if pm is None else jnp.maximum(pm, t_pm)
        pmax.append(pm)

    recips = []
    for it in range(ni):
        ii = it * _TI
        amax = jnp.where(pmax[it] > 0.5 * _NEG, pmax[it], 0.0)
        den = None
        for jt in range(nj):
            jj = jt * _TJ
            e = jnp.exp(s_scr[jj:jj + _TJ, ii:ii + _TI] - amax)
            s_scr[jj:jj + _TJ, ii:ii + _TI] = e
            t_den = jnp.sum(e, axis=0, keepdims=True)
            den = t_den if den is None else den + t_den
        recips.append(1.0 / (den + 1e-16))
    recip = jnp.concatenate(recips, axis=1)

    ex = s_scr[...]
    out_t = jnp.dot(xlt, ex, preferred_element_type=jnp.float32)
    out_ref[0, 0] = out_t * recip + bias_ref[0]


def kernel(x, adj, Wl, Wr, att, bias):
    b, n, in_dim = x.shape
    heads, dh = att.shape

    xt = x.transpose(0, 2, 1)
    adj8 = (adj != 0).astype(jnp.int8)
    wl = Wl.reshape(in_dim, heads, dh).transpose(1, 0, 2)
    wlt = Wl.reshape(in_dim, heads, dh).transpose(1, 2, 0)
    wrt = Wr.reshape(in_dim, heads, dh).transpose(1, 2, 0)
    attr = att.reshape(heads, 1, dh)
    attc = att.reshape(heads, dh, 1)
    biasc = bias.reshape(heads, dh, 1)

    out = pl.pallas_call(
        _gat_body,
        grid=(b, heads),
        in_specs=[
            pl.BlockSpec((1, n, in_dim), lambda bb, h: (bb, 0, 0)),
            pl.BlockSpec((1, in_dim, n), lambda bb, h: (bb, 0, 0)),
            pl.BlockSpec((1, n, n), lambda bb, h: (bb, 0, 0)),
            pl.BlockSpec((1, in_dim, dh), lambda bb, h: (h, 0, 0)),
            pl.BlockSpec((1, dh, in_dim), lambda bb, h: (h, 0, 0)),
            pl.BlockSpec((1, dh, in_dim), lambda bb, h: (h, 0, 0)),
            pl.BlockSpec((1, 1, dh), lambda bb, h: (h, 0, 0)),
            pl.BlockSpec((1, dh, 1), lambda bb, h: (h, 0, 0)),
            pl.BlockSpec((1, dh, 1), lambda bb, h: (h, 0, 0)),
        ],
        out_specs=pl.BlockSpec((1, 1, dh, n), lambda bb, h: (bb, h, 0, 0)),
        out_shape=jax.ShapeDtypeStruct((b, heads, dh, n), jnp.float32),
        scratch_shapes=[pltpu.VMEM((n, n), jnp.float32)],
        compiler_params=pltpu.CompilerParams(
            dimension_semantics=("parallel", "parallel")),
    )(x, xt, adj8, wl, wlt, wrt, attr, attc, biasc)

    return out.transpose(0, 3, 1, 2).reshape(b, n, heads * dh)


# speedup vs baseline: 6.4882x; 1.5119x over previous
import jax
import jax.numpy as jnp
from jax import lax
from jax.experimental import pallas as pl
from jax.experimental.pallas import tpu as pltpu

_NEG = -1e30
_TJ = 128
_TI = 128


def _gat_body(x_ref, xt_ref, adj_ref, wl_ref, wlt_ref, wrt_ref, att_ref,
              attc_ref, bias_ref, out_ref, s_scr):
    n = x_ref.shape[1]
    dh = wl_ref.shape[2]
    x = x_ref[0]
    xt = xt_ref[0]
    wl = wl_ref[0]
    wlt = wlt_ref[0]
    wrt = wrt_ref[0]
    att = att_ref[0]
    attc = attc_ref[0]

    xl = jnp.dot(x, wl, preferred_element_type=jnp.float32)
    xlt = jnp.dot(wlt, xt, preferred_element_type=jnp.float32)
    xrat = jnp.dot(wrt, xt, preferred_element_type=jnp.float32)

    xla = (xl * (0.4 * att)).astype(jnp.bfloat16)
    xrab = (xrat * (0.4 * attc)).astype(jnp.bfloat16)
    al2 = 1.5 * jnp.sum(xla, axis=1, keepdims=True)
    ar2 = 1.5 * jnp.sum(xrab, axis=0, keepdims=True)

    nj = n // _TJ
    ni = n // _TI

    pmax = []
    for it in range(ni):
        ii = it * _TI
        pm = None
        for jt in range(nj):
            jj = jt * _TJ
            accb = jnp.zeros((_TJ, _TI), jnp.bfloat16)
            for d in range(dh):
                t = xla[jj:jj + _TJ, d:d + 1] + xrab[d:d + 1, ii:ii + _TI]
                accb = accb + jnp.abs(t) * jnp.sign(att[0, d]).astype(jnp.bfloat16)
            acc = al2[jj:jj + _TJ] + ar2[:, ii:ii + _TI] + accb.astype(jnp.float32)
            m = adj_ref[0, jj:jj + _TJ, ii:ii + _TI] != 0
            acc = jnp.where(m, acc, _NEG)
            s_scr[jj:jj + _TJ, ii:ii + _TI] = acc
            t_pm = jnp.max(acc, axis=0, keepdims=True)
            pm = t_pm if pm is None else jnp.maximum(pm, t_pm)
        pmax.append(pm)

    recips = []
    for it in range(ni):
        ii = it * _TI
        amax = jnp.where(pmax[it] > 0.5 * _NEG, pmax[it], 0.0)
        den = None
        for jt in range(nj):
            jj = jt * _TJ
            e = jnp.exp(s_scr[jj:jj + _TJ, ii:ii + _TI] - amax)
            s_scr[jj:jj + _TJ, ii:ii + _TI] = e
            t_den = jnp.sum(e, axis=0, keepdims=True)
            den = t_den if den is None else den + t_den
        recips.append(1.0 / (den + 1e-16))
    recip = jnp.concatenate(recips, axis=1)

    ex = s_scr[...]
    out_t = jnp.dot(xlt, ex, preferred_element_type=jnp.float32)
    out_ref[0, 0] = out_t * recip + bias_ref[0]


def kernel(x, adj, Wl, Wr, att, bias):
    b, n, in_dim = x.shape
    heads, dh = att.shape

    xt = x.transpose(0, 2, 1)
    adj8 = (adj != 0).astype(jnp.int8)
    wl = Wl.reshape(in_dim, heads, dh).transpose(1, 0, 2)
    wlt = Wl.reshape(in_dim, heads, dh).transpose(1, 2, 0)
    wrt = Wr.reshape(in_dim, heads, dh).transpose(1, 2, 0)
    attr = att.reshape(heads, 1, dh)
    attc = att.reshape(heads, dh, 1)
    biasc = bias.reshape(heads, dh, 1)

    out = pl.pallas_call(
        _gat_body,
        grid=(b, heads),
        in_specs=[
            pl.BlockSpec((1, n, in_dim), lambda bb, h: (bb, 0, 0)),
            pl.BlockSpec((1, in_dim, n), lambda bb, h: (bb, 0, 0)),
            pl.BlockSpec((1, n, n), lambda bb, h: (bb, 0, 0)),
            pl.BlockSpec((1, in_dim, dh), lambda bb, h: (h, 0, 0)),
            pl.BlockSpec((1, dh, in_dim), lambda bb, h: (h, 0, 0)),
            pl.BlockSpec((1, dh, in_dim), lambda bb, h: (h, 0, 0)),
            pl.BlockSpec((1, 1, dh), lambda bb, h: (h, 0, 0)),
            pl.BlockSpec((1, dh, 1), lambda bb, h: (h, 0, 0)),
            pl.BlockSpec((1, dh, 1), lambda bb, h: (h, 0, 0)),
        ],
        out_specs=pl.BlockSpec((1, 1, dh, n), lambda bb, h: (bb, h, 0, 0)),
        out_shape=jax.ShapeDtypeStruct((b, heads, dh, n), jnp.float32),
        scratch_shapes=[pltpu.VMEM((n, n), jnp.float32)],
        compiler_params=pltpu.CompilerParams(
            dimension_semantics=("parallel", "parallel")),
    )(x, xt, adj8, wl, wlt, wrt, attr, attc, biasc)

    return out.transpose(0, 3, 1, 2).reshape(b, n, heads * dh)
